# baseline (speedup 1.0000x reference)
import functools

import jax
import jax.numpy as jnp
from jax import lax
from jax.experimental import pallas as pl
from jax.experimental.pallas import tpu as pltpu
from jax.experimental.pallas import tpu_sc as plsc

_R = 4 * 4096
_C = 1024
_NC = 2
_NS = 16
_NW = _NC * _NS
_RPW = _R // _NW
_CH = 16


def _sc_body(x0h, x1h, x2h, m0h, m1h, thh,
             out0h, out1h, fusedh,
             m0_v, m1_v, th_v, buf0, buf1, buf2, sem):
    wid = lax.axis_index("s") * _NC + lax.axis_index("c")
    base = wid * _RPW

    pltpu.sync_copy(m0h.at[pl.ds(base, _RPW)], m0_v)
    pltpu.sync_copy(m1h.at[pl.ds(base, _RPW)], m1_v)
    pltpu.sync_copy(thh, th_v)
    thvec = th_v[...]
    theta = thvec[0]
    miu = thvec[1]

    def chunk(ci, carry):
        rowbase = base + ci * _CH
        mvec0 = m0_v[pl.ds(ci * _CH, _CH)]
        mvec1 = m1_v[pl.ds(ci * _CH, _CH)]
        for r in range(_CH):
            m0s = mvec0[r]
            m1s = mvec1[r]
            row = rowbase + r

            @pl.when(m0s >= theta)
            def _():
                pltpu.async_copy(x0h.at[pl.ds(row, 1)], buf0.at[pl.ds(r, 1)], sem)

            @pl.when(m0s < theta)
            def _():
                pltpu.async_copy(x1h.at[pl.ds(row, 1)], buf0.at[pl.ds(r, 1)], sem)

            @pl.when(m1s >= theta)
            def _():
                pltpu.async_copy(x1h.at[pl.ds(row, 1)], buf1.at[pl.ds(r, 1)], sem)

            @pl.when(m1s < theta)
            def _():
                pltpu.async_copy(x0h.at[pl.ds(row, 1)], buf1.at[pl.ds(r, 1)], sem)

            c1 = m1s >= miu
            c0 = jnp.logical_and(m0s >= miu, jnp.logical_not(c1))
            c2 = jnp.logical_and(m1s < miu, m0s < miu)

            @pl.when(c1)
            def _():
                pltpu.async_copy(x1h.at[pl.ds(row, 1)], buf2.at[pl.ds(r, 1)], sem)

            @pl.when(c0)
            def _():
                pltpu.async_copy(x0h.at[pl.ds(row, 1)], buf2.at[pl.ds(r, 1)], sem)

            @pl.when(c2)
            def _():
                pltpu.async_copy(x2h.at[pl.ds(row, 1)], buf2.at[pl.ds(r, 1)], sem)

        for r in range(_CH):
            pltpu.make_async_copy(x0h.at[pl.ds(0, 1)], buf0.at[pl.ds(r, 1)], sem).wait()
            pltpu.make_async_copy(x0h.at[pl.ds(0, 1)], buf1.at[pl.ds(r, 1)], sem).wait()
            pltpu.make_async_copy(x0h.at[pl.ds(0, 1)], buf2.at[pl.ds(r, 1)], sem).wait()

        pltpu.sync_copy(buf0, out0h.at[pl.ds(rowbase, _CH)])
        pltpu.sync_copy(buf1, out1h.at[pl.ds(rowbase, _CH)])
        pltpu.sync_copy(buf2, fusedh.at[pl.ds(rowbase, _CH)])
        return carry

    lax.fori_loop(0, _RPW // _CH, chunk, 0)


_sc_call = functools.partial(
    pl.kernel,
    out_type=[jax.ShapeDtypeStruct((_R, _C), jnp.float32)] * 3,
    mesh=plsc.VectorSubcoreMesh(core_axis_name="c", subcore_axis_name="s"),
    scratch_types=[
        pltpu.VMEM((_RPW,), jnp.float32),
        pltpu.VMEM((_RPW,), jnp.float32),
        pltpu.VMEM((16,), jnp.float32),
        pltpu.VMEM((_CH, _C), jnp.float32),
        pltpu.VMEM((_CH, _C), jnp.float32),
        pltpu.VMEM((_CH, _C), jnp.float32),
        pltpu.SemaphoreType.DMA,
    ],
)(_sc_body)


def kernel(x0, x1, x2, mask0, mask1, mask_threshold_theta, mask_threshold_miu):
    B, N, C = x0.shape
    x0f = x0.reshape(_R, C)
    x1f = x1.reshape(_R, C)
    x2f = x2.reshape(_R, C)
    m0f = mask0.reshape(_R)
    m1f = mask1.reshape(_R)
    th = jnp.zeros((16,), jnp.float32)
    th = th.at[0].set(jnp.asarray(mask_threshold_theta, jnp.float32))
    th = th.at[1].set(jnp.asarray(mask_threshold_miu, jnp.float32))

    out0, out1, fused = _sc_call(x0f, x1f, x2f, m0f, m1f, th)
    return (out0.reshape(B, N, C), out1.reshape(B, N, C),
            fused.reshape(B, N, C))


# speedup vs baseline: 1.0606x; 1.0606x over previous
import functools

import jax
import jax.numpy as jnp
from jax import lax
from jax.experimental import pallas as pl
from jax.experimental.pallas import tpu as pltpu
from jax.experimental.pallas import tpu_sc as plsc

_R = 4 * 4096
_C = 1024
_NC = 2
_NS = 16
_NW = _NC * _NS
_RPW = _R // _NW
_CH = 16
_NCHUNK = _RPW // _CH


def _sc_body(x0h, x1h, x2h, m0h, m1h, thh,
             out0h, out1h, fusedh,
             m0_v, m1_v, th_v,
             a0, b0, c0, a1, b1, c1,
             gsem0, gsem1, ssem0, ssem1):
    wid = lax.axis_index("s") * _NC + lax.axis_index("c")
    base = wid * _RPW

    pltpu.sync_copy(m0h.at[pl.ds(base, _RPW)], m0_v)
    pltpu.sync_copy(m1h.at[pl.ds(base, _RPW)], m1_v)
    pltpu.sync_copy(thh, th_v)
    thvec = th_v[...]
    theta = thvec[0]
    miu = thvec[1]

    slot0 = (a0, b0, c0)
    slot1 = (a1, b1, c1)

    def issue_gathers(c, bufs, gsem):
        rowbase = base + c * _CH
        mvec0 = m0_v[pl.ds(c * _CH, _CH)]
        mvec1 = m1_v[pl.ds(c * _CH, _CH)]
        buf0, buf1, buf2 = bufs
        for r in range(_CH):
            m0s = mvec0[r]
            m1s = mvec1[r]
            row = rowbase + r

            @pl.when(m0s >= theta)
            def _():
                pltpu.async_copy(x0h.at[pl.ds(row, 1)], buf0.at[pl.ds(r, 1)], gsem)

            @pl.when(m0s < theta)
            def _():
                pltpu.async_copy(x1h.at[pl.ds(row, 1)], buf0.at[pl.ds(r, 1)], gsem)

            @pl.when(m1s >= theta)
            def _():
                pltpu.async_copy(x1h.at[pl.ds(row, 1)], buf1.at[pl.ds(r, 1)], gsem)

            @pl.when(m1s < theta)
            def _():
                pltpu.async_copy(x0h.at[pl.ds(row, 1)], buf1.at[pl.ds(r, 1)], gsem)

            f1 = m1s >= miu
            f0 = jnp.logical_and(m0s >= miu, jnp.logical_not(f1))
            f2 = jnp.logical_and(m1s < miu, m0s < miu)

            @pl.when(f1)
            def _():
                pltpu.async_copy(x1h.at[pl.ds(row, 1)], buf2.at[pl.ds(r, 1)], gsem)

            @pl.when(f0)
            def _():
                pltpu.async_copy(x0h.at[pl.ds(row, 1)], buf2.at[pl.ds(r, 1)], gsem)

            @pl.when(f2)
            def _():
                pltpu.async_copy(x2h.at[pl.ds(row, 1)], buf2.at[pl.ds(r, 1)], gsem)

    def drain_gathers(bufs, gsem):
        for buf in bufs:
            for r in range(_CH):
                pltpu.make_async_copy(x0h.at[pl.ds(0, 1)], buf.at[pl.ds(r, 1)], gsem).wait()

    def issue_scatters(c, bufs, ssem):
        rowbase = base + c * _CH
        pltpu.async_copy(bufs[0], out0h.at[pl.ds(rowbase, _CH)], ssem)
        pltpu.async_copy(bufs[1], out1h.at[pl.ds(rowbase, _CH)], ssem)
        pltpu.async_copy(bufs[2], fusedh.at[pl.ds(rowbase, _CH)], ssem)

    def drain_scatters(bufs, ssem):
        pltpu.make_async_copy(bufs[0], out0h.at[pl.ds(0, _CH)], ssem).wait()
        pltpu.make_async_copy(bufs[1], out1h.at[pl.ds(0, _CH)], ssem).wait()
        pltpu.make_async_copy(bufs[2], fusedh.at[pl.ds(0, _CH)], ssem).wait()

    issue_gathers(0, slot0, gsem0)
    issue_gathers(1, slot1, gsem1)
    drain_gathers(slot0, gsem0)
    issue_scatters(0, slot0, ssem0)

    def group(g, carry):
        ca = 1 + 2 * g
        cb = ca + 1

        drain_scatters(slot0, ssem0)
        issue_gathers(ca + 1, slot0, gsem0)
        drain_gathers(slot1, gsem1)
        issue_scatters(ca, slot1, ssem1)

        drain_scatters(slot1, ssem1)
        issue_gathers(cb + 1, slot1, gsem1)
        drain_gathers(slot0, gsem0)
        issue_scatters(cb, slot0, ssem0)
        return carry

    lax.fori_loop(0, (_NCHUNK - 2) // 2, group, 0)

    drain_scatters(slot0, ssem0)
    drain_gathers(slot1, gsem1)
    issue_scatters(_NCHUNK - 1, slot1, ssem1)
    drain_scatters(slot1, ssem1)


_sc_call = functools.partial(
    pl.kernel,
    out_type=[jax.ShapeDtypeStruct((_R, _C), jnp.float32)] * 3,
    mesh=plsc.VectorSubcoreMesh(core_axis_name="c", subcore_axis_name="s"),
    scratch_types=[
        pltpu.VMEM((_RPW,), jnp.float32),
        pltpu.VMEM((_RPW,), jnp.float32),
        pltpu.VMEM((16,), jnp.float32),
        pltpu.VMEM((_CH, _C), jnp.float32),
        pltpu.VMEM((_CH, _C), jnp.float32),
        pltpu.VMEM((_CH, _C), jnp.float32),
        pltpu.VMEM((_CH, _C), jnp.float32),
        pltpu.VMEM((_CH, _C), jnp.float32),
        pltpu.VMEM((_CH, _C), jnp.float32),
        pltpu.SemaphoreType.DMA,
        pltpu.SemaphoreType.DMA,
        pltpu.SemaphoreType.DMA,
        pltpu.SemaphoreType.DMA,
    ],
)(_sc_body)


def kernel(x0, x1, x2, mask0, mask1, mask_threshold_theta, mask_threshold_miu):
    B, N, C = x0.shape
    x0f = x0.reshape(_R, C)
    x1f = x1.reshape(_R, C)
    x2f = x2.reshape(_R, C)
    m0f = mask0.reshape(_R)
    m1f = mask1.reshape(_R)
    th = jnp.zeros((16,), jnp.float32)
    th = th.at[0].set(jnp.asarray(mask_threshold_theta, jnp.float32))
    th = th.at[1].set(jnp.asarray(mask_threshold_miu, jnp.float32))

    out0, out1, fused = _sc_call(x0f, x1f, x2f, m0f, m1f, th)
    return (out0.reshape(B, N, C), out1.reshape(B, N, C),
            fused.reshape(B, N, C))


# speedup vs baseline: 1.2787x; 1.2057x over previous
import functools

import jax
import jax.numpy as jnp
from jax import lax
from jax.experimental import pallas as pl
from jax.experimental.pallas import tpu as pltpu
from jax.experimental.pallas import tpu_sc as plsc

_R = 4 * 4096
_C = 1024
_NC = 2
_NS = 16
_NW = _NC * _NS
_RPW = _R // _NW
_CH = 32
_NCHUNK = _RPW // _CH


def _sc_fused_body(x0h, x1h, x2h, m0h, m1h, thh, fusedh,
                   m0_v, m1_v, th_v, buf0, buf1, gsem0, gsem1, ssem0, ssem1):
    wid = lax.axis_index("s") * _NC + lax.axis_index("c")
    base = wid * _RPW

    pltpu.sync_copy(m0h.at[pl.ds(base, _RPW)], m0_v)
    pltpu.sync_copy(m1h.at[pl.ds(base, _RPW)], m1_v)
    pltpu.sync_copy(thh, th_v)
    thvec = th_v[...]
    miu = thvec[1]

    def issue_gathers(c, buf, gsem):
        rowbase = base + c * _CH
        for k in range(_CH // 16):
            mvec0 = m0_v[pl.ds(c * _CH + k * 16, 16)]
            mvec1 = m1_v[pl.ds(c * _CH + k * 16, 16)]
            for r in range(16):
                m0s = mvec0[r]
                m1s = mvec1[r]
                row = rowbase + k * 16 + r
                dst = buf.at[pl.ds(k * 16 + r, 1)]

                f1 = m1s >= miu
                f0 = jnp.logical_and(m0s >= miu, jnp.logical_not(f1))
                f2 = jnp.logical_and(m1s < miu, m0s < miu)

                @pl.when(f1)
                def _():
                    pltpu.async_copy(x1h.at[pl.ds(row, 1)], dst, gsem)

                @pl.when(f0)
                def _():
                    pltpu.async_copy(x0h.at[pl.ds(row, 1)], dst, gsem)

                @pl.when(f2)
                def _():
                    pltpu.async_copy(x2h.at[pl.ds(row, 1)], dst, gsem)

    def drain_gathers(buf, gsem):
        for r in range(_CH):
            pltpu.make_async_copy(x0h.at[pl.ds(0, 1)], buf.at[pl.ds(r, 1)], gsem).wait()

    def issue_scatter(c, buf, ssem):
        pltpu.async_copy(buf, fusedh.at[pl.ds(base + c * _CH, _CH)], ssem)

    def drain_scatter(buf, ssem):
        pltpu.make_async_copy(buf, fusedh.at[pl.ds(0, _CH)], ssem).wait()

    issue_gathers(0, buf0, gsem0)
    issue_gathers(1, buf1, gsem1)
    drain_gathers(buf0, gsem0)
    issue_scatter(0, buf0, ssem0)

    def group(g, carry):
        ca = 1 + 2 * g
        cb = ca + 1

        drain_scatter(buf0, ssem0)
        issue_gathers(ca + 1, buf0, gsem0)
        drain_gathers(buf1, gsem1)
        issue_scatter(ca, buf1, ssem1)

        drain_scatter(buf1, ssem1)
        issue_gathers(cb + 1, buf1, gsem1)
        drain_gathers(buf0, gsem0)
        issue_scatter(cb, buf0, ssem0)
        return carry

    lax.fori_loop(0, (_NCHUNK - 2) // 2, group, 0)

    drain_scatter(buf0, ssem0)
    drain_gathers(buf1, gsem1)
    issue_scatter(_NCHUNK - 1, buf1, ssem1)
    drain_scatter(buf1, ssem1)


_sc_fused = functools.partial(
    pl.kernel,
    out_type=jax.ShapeDtypeStruct((_R, _C), jnp.float32),
    mesh=plsc.VectorSubcoreMesh(core_axis_name="c", subcore_axis_name="s"),
    scratch_types=[
        pltpu.VMEM((_RPW,), jnp.float32),
        pltpu.VMEM((_RPW,), jnp.float32),
        pltpu.VMEM((16,), jnp.float32),
        pltpu.VMEM((_CH, _C), jnp.float32),
        pltpu.VMEM((_CH, _C), jnp.float32),
        pltpu.SemaphoreType.DMA,
        pltpu.SemaphoreType.DMA,
        pltpu.SemaphoreType.DMA,
        pltpu.SemaphoreType.DMA,
    ],
)(_sc_fused_body)


_BLOCK_ROWS = 512


def _tc_body(theta_ref, m0_ref, m1_ref, x0_ref, x1_ref, out0_ref, out1_ref):
    theta = theta_ref[0]
    m0 = m0_ref[...]
    m1 = m1_ref[...]
    x0 = x0_ref[...]
    x1 = x1_ref[...]
    out0_ref[...] = jnp.where(m0 >= theta, x0, x1)
    out1_ref[...] = jnp.where(m1 >= theta, x1, x0)


def _tc_call(theta, m0f, m1f, x0f, x1f):
    grid = (_R // _BLOCK_ROWS,)
    row_block = pl.BlockSpec((_BLOCK_ROWS, _C), lambda i: (i, 0))
    mask_block = pl.BlockSpec((_BLOCK_ROWS, 1), lambda i: (i, 0))
    scalar_spec = pl.BlockSpec(memory_space=pltpu.SMEM)
    return pl.pallas_call(
        _tc_body,
        grid=grid,
        in_specs=[scalar_spec, mask_block, mask_block, row_block, row_block],
        out_specs=[row_block, row_block],
        out_shape=[jax.ShapeDtypeStruct((_R, _C), jnp.float32)] * 2,
    )(theta, m0f, m1f, x0f, x1f)


def kernel(x0, x1, x2, mask0, mask1, mask_threshold_theta, mask_threshold_miu):
    B, N, C = x0.shape
    x0f = x0.reshape(_R, C)
    x1f = x1.reshape(_R, C)
    x2f = x2.reshape(_R, C)
    m0f = mask0.reshape(_R)
    m1f = mask1.reshape(_R)
    th = jnp.zeros((16,), jnp.float32)
    th = th.at[0].set(jnp.asarray(mask_threshold_theta, jnp.float32))
    th = th.at[1].set(jnp.asarray(mask_threshold_miu, jnp.float32))
    theta = jnp.asarray(mask_threshold_theta, jnp.float32).reshape(1)

    fused = _sc_fused(x0f, x1f, x2f, m0f, m1f, th)
    out0, out1 = _tc_call(theta, m0f.reshape(_R, 1), m1f.reshape(_R, 1),
                          x0f, x1f)

    return (out0.reshape(B, N, C), out1.reshape(B, N, C),
            fused.reshape(B, N, C))


# speedup vs baseline: 1.3972x; 1.0927x over previous
import functools

import jax
import jax.numpy as jnp
from jax import lax
from jax.experimental import pallas as pl
from jax.experimental.pallas import tpu as pltpu
from jax.experimental.pallas import tpu_sc as plsc

_R = 4 * 4096
_C = 1024
_NC = 2
_NS = 16
_NW = _NC * _NS
_RPW = _R // _NW
_CH = 32
_NCHUNK = _RPW // _CH


def _sc_fused_body(x0h, x1h, x2h, m0h, m1h, thh, fusedh,
                   m0_v, m1_v, th_v, buf0, buf1, gsem0, gsem1, ssem0, ssem1):
    wid = lax.axis_index("s") * _NC + lax.axis_index("c")
    base = wid * _RPW

    pltpu.sync_copy(m0h.at[pl.ds(base, _RPW)], m0_v)
    pltpu.sync_copy(m1h.at[pl.ds(base, _RPW)], m1_v)
    pltpu.sync_copy(thh, th_v)
    thvec = th_v[...]
    miu = thvec[8]

    def issue_gathers(c, buf, gsem):
        rowbase = base + c * _CH
        for k in range(_CH // 16):
            mvec0 = m0_v[pl.ds(c * _CH + k * 16, 16)]
            mvec1 = m1_v[pl.ds(c * _CH + k * 16, 16)]
            selvec = jnp.where(mvec1 >= miu, 1,
                               jnp.where(mvec0 >= miu, 0, 2)).astype(jnp.int32)
            for r in range(16):
                s = selvec[r]
                row = rowbase + k * 16 + r
                dst = buf.at[pl.ds(k * 16 + r, 1)]

                @pl.when(s == 1)
                def _():
                    pltpu.async_copy(x1h.at[pl.ds(row, 1)], dst, gsem)

                @pl.when(s == 0)
                def _():
                    pltpu.async_copy(x0h.at[pl.ds(row, 1)], dst, gsem)

                @pl.when(s == 2)
                def _():
                    pltpu.async_copy(x2h.at[pl.ds(row, 1)], dst, gsem)

    def drain_gathers(buf, gsem):
        for r in range(_CH):
            pltpu.make_async_copy(x0h.at[pl.ds(0, 1)], buf.at[pl.ds(r, 1)], gsem).wait()

    def issue_scatter(c, buf, ssem):
        pltpu.async_copy(buf, fusedh.at[pl.ds(base + c * _CH, _CH)], ssem)

    def drain_scatter(buf, ssem):
        pltpu.make_async_copy(buf, fusedh.at[pl.ds(0, _CH)], ssem).wait()

    issue_gathers(0, buf0, gsem0)
    issue_gathers(1, buf1, gsem1)
    drain_gathers(buf0, gsem0)
    issue_scatter(0, buf0, ssem0)

    def group(g, carry):
        ca = 1 + 2 * g
        cb = ca + 1

        drain_scatter(buf0, ssem0)
        issue_gathers(ca + 1, buf0, gsem0)
        drain_gathers(buf1, gsem1)
        issue_scatter(ca, buf1, ssem1)

        drain_scatter(buf1, ssem1)
        issue_gathers(cb + 1, buf1, gsem1)
        drain_gathers(buf0, gsem0)
        issue_scatter(cb, buf0, ssem0)
        return carry

    lax.fori_loop(0, (_NCHUNK - 2) // 2, group, 0)

    drain_scatter(buf0, ssem0)
    drain_gathers(buf1, gsem1)
    issue_scatter(_NCHUNK - 1, buf1, ssem1)
    drain_scatter(buf1, ssem1)


_sc_fused = functools.partial(
    pl.kernel,
    out_type=jax.ShapeDtypeStruct((_R, _C), jnp.float32),
    mesh=plsc.VectorSubcoreMesh(core_axis_name="c", subcore_axis_name="s"),
    scratch_types=[
        pltpu.VMEM((_RPW,), jnp.float32),
        pltpu.VMEM((_RPW,), jnp.float32),
        pltpu.VMEM((16,), jnp.float32),
        pltpu.VMEM((_CH, _C), jnp.float32),
        pltpu.VMEM((_CH, _C), jnp.float32),
        pltpu.SemaphoreType.DMA,
        pltpu.SemaphoreType.DMA,
        pltpu.SemaphoreType.DMA,
        pltpu.SemaphoreType.DMA,
    ],
)(_sc_fused_body)


_BLOCK_ROWS = 1024


def _tc_body(theta_ref, m0_ref, m1_ref, x0_ref, x1_ref, out0_ref, out1_ref):
    theta = theta_ref[0]
    m0 = m0_ref[0, 0, :].reshape(_BLOCK_ROWS, 1)
    m1 = m1_ref[0, 0, :].reshape(_BLOCK_ROWS, 1)
    x0 = x0_ref[...]
    x1 = x1_ref[...]
    out0_ref[...] = jnp.where(m0 >= theta, x0, x1)
    out1_ref[...] = jnp.where(m1 >= theta, x1, x0)


def _tc_call(theta, m0f, m1f, x0f, x1f):
    grid = (_R // _BLOCK_ROWS,)
    row_block = pl.BlockSpec((_BLOCK_ROWS, _C), lambda i: (i, 0))
    mask_block = pl.BlockSpec((1, 1, _BLOCK_ROWS), lambda i: (i, 0, 0))
    scalar_spec = pl.BlockSpec(memory_space=pltpu.SMEM)
    return pl.pallas_call(
        _tc_body,
        grid=grid,
        in_specs=[scalar_spec, mask_block, mask_block, row_block, row_block],
        out_specs=[row_block, row_block],
        out_shape=[jax.ShapeDtypeStruct((_R, _C), jnp.float32)] * 2,
    )(theta, m0f, m1f, x0f, x1f)


def kernel(x0, x1, x2, mask0, mask1, mask_threshold_theta, mask_threshold_miu):
    B, N, C = x0.shape
    x0f = x0.reshape(_R, C)
    x1f = x1.reshape(_R, C)
    x2f = x2.reshape(_R, C)
    m0f = mask0.reshape(_R)
    m1f = mask1.reshape(_R)
    theta_s = jnp.asarray(mask_threshold_theta, jnp.float32).reshape(1)
    miu_s = jnp.asarray(mask_threshold_miu, jnp.float32).reshape(1)
    th = jnp.concatenate([jnp.broadcast_to(theta_s, (8,)),
                          jnp.broadcast_to(miu_s, (8,))])

    fused = _sc_fused(x0f, x1f, x2f, m0f, m1f, th)
    out0, out1 = _tc_call(theta_s,
                          m0f.reshape(_R // _BLOCK_ROWS, 1, _BLOCK_ROWS),
                          m1f.reshape(_R // _BLOCK_ROWS, 1, _BLOCK_ROWS),
                          x0f, x1f)

    return (out0.reshape(B, N, C), out1.reshape(B, N, C),
            fused.reshape(B, N, C))
